# baseline (speedup 1.0000x reference)
import functools

import jax
import jax.numpy as jnp
import numpy as np
from jax import lax
from jax.experimental import pallas as pl
from jax.experimental.pallas import tpu as pltpu
from jax.experimental.pallas import tpu_sc as plsc

NC = 2
NS = 16
NW = NC * NS
ROW = 128


def _worker_split(total_rows):
    base = total_rows // NW
    extra = total_rows % NW
    return base, extra


def _worker_slabs(idx_rows, total_rows):
    base, extra = _worker_split(total_rows)
    rpw8 = -(-(base + 1) // 8) * 8
    starts = np.array([w * base + min(w, extra) for w in range(NW)])
    rows = np.minimum(starts[:, None] + np.arange(rpw8)[None, :], total_rows - 1)
    return jnp.take(idx_rows, jnp.asarray(rows), axis=0), rpw8


def _sc_gather(table, idx_slabs, rpw8, total_rows):
    n, d = table.shape
    base, extra = _worker_split(total_rows)
    mesh = plsc.VectorSubcoreMesh(
        core_axis_name="c", subcore_axis_name="s", num_cores=NC, num_subcores=NS
    )

    nbuf = 4

    @functools.partial(
        pl.kernel,
        out_type=jax.ShapeDtypeStruct((total_rows * ROW, d), jnp.float32),
        mesh=mesh,
        scratch_types=[
            pltpu.VMEM((rpw8, ROW), jnp.int32),
            [pltpu.VMEM((ROW, d), jnp.float32) for _ in range(nbuf)],
            [pltpu.SemaphoreType.DMA for _ in range(nbuf)],
        ],
    )
    def k(table_hbm, idx_hbm, out_hbm, idx_v, bufs, gsems):
        c = lax.axis_index("c")
        s = lax.axis_index("s")
        w = s * NC + c
        start = w * base + jnp.minimum(w, extra)
        cnt = base + jnp.where(w < extra, 1, 0)
        pltpu.sync_copy(idx_hbm.at[w], idx_v)

        for p in range(nbuf):
            @pl.when(p < cnt)
            def _():
                pltpu.async_copy(table_hbm.at[idx_v.at[p]], bufs[p], gsems[p])

        def body(j, _):
            for p in range(nbuf):
                @pl.when(lax.rem(j, nbuf) == p)
                def _():
                    pltpu.make_async_copy(
                        table_hbm.at[idx_v.at[0]], bufs[p], gsems[p]).wait()
                    pltpu.sync_copy(bufs[p], out_hbm.at[pl.ds((start + j) * ROW, ROW)])

                    @pl.when(j + nbuf < cnt)
                    def _():
                        pltpu.async_copy(table_hbm.at[idx_v.at[j + nbuf]],
                                         bufs[p], gsems[p])
            return 0

        lax.fori_loop(0, cnt, body, 0)

    return k(table, idx_slabs)


def _ln_affine(h, gamma, beta, eps=1e-5):
    mu = jnp.mean(h, axis=-1, keepdims=True)
    xc = h - mu
    var = jnp.mean(xc * xc, axis=-1, keepdims=True)
    return xc * lax.rsqrt(var + eps) * gamma + beta


def _edge_mlp_body(gs_ref, gr_ref, e_ref, w0_ref, b0_ref, w1_ref, b1_ref,
                   w2_ref, b2_ref, g_ref, bt_ref, mlp_ref, edge_ref):
    bf = jnp.bfloat16
    s = gs_ref[0].astype(bf)
    r = gr_ref[0].astype(bf)
    e = e_ref[...]
    w0 = w0_ref[...].astype(bf)
    d = e.shape[1]
    h = (jnp.dot(s, w0[0:d], preferred_element_type=jnp.float32)
         + jnp.dot(r, w0[d:2 * d], preferred_element_type=jnp.float32)
         + jnp.dot(e.astype(bf), w0[2 * d:3 * d], preferred_element_type=jnp.float32)
         + b0_ref[...])
    h = jnp.maximum(h, 0.0)
    h = jnp.maximum(jnp.dot(h.astype(bf), w1_ref[...].astype(bf),
                            preferred_element_type=jnp.float32)
                    + b1_ref[...], 0.0)
    h = jnp.dot(h.astype(bf), w2_ref[...].astype(bf),
                preferred_element_type=jnp.float32) + b2_ref[...]
    h = _ln_affine(h, g_ref[...], bt_ref[...])
    mlp_ref[...] = h
    edge_ref[...] = h + e


def _tc_edge_mlp(gathered, edge_features, edge_params, block_e):
    e_total, d = edge_features.shape
    nb = e_total // block_e
    w0, b0, w1, b1, w2, b2, gamma, beta = edge_params
    vec = lambda v: v.reshape(1, d)
    g3 = gathered.reshape(2, e_total, d)

    out = pl.pallas_call(
        _edge_mlp_body,
        grid=(nb,),
        in_specs=[
            pl.BlockSpec((1, block_e, d), lambda i: (0, i, 0)),
            pl.BlockSpec((1, block_e, d), lambda i: (1, i, 0)),
            pl.BlockSpec((block_e, d), lambda i: (i, 0)),
            pl.BlockSpec((3 * d, d), lambda i: (0, 0)),
            pl.BlockSpec((1, d), lambda i: (0, 0)),
            pl.BlockSpec((d, d), lambda i: (0, 0)),
            pl.BlockSpec((1, d), lambda i: (0, 0)),
            pl.BlockSpec((d, d), lambda i: (0, 0)),
            pl.BlockSpec((1, d), lambda i: (0, 0)),
            pl.BlockSpec((1, d), lambda i: (0, 0)),
            pl.BlockSpec((1, d), lambda i: (0, 0)),
        ],
        out_specs=[
            pl.BlockSpec((block_e, d), lambda i: (i, 0)),
            pl.BlockSpec((block_e, d), lambda i: (i, 0)),
        ],
        out_shape=[
            jax.ShapeDtypeStruct((e_total, d), jnp.float32),
            jax.ShapeDtypeStruct((e_total, d), jnp.float32),
        ],
        compiler_params=pltpu.CompilerParams(
            dimension_semantics=("arbitrary",),
        ),
    )(g3, g3, edge_features, w0, vec(b0), w1, vec(b1), w2, vec(b2),
      vec(gamma), vec(beta))
    return out


def _sc_scatter(values, idx_slabs, rpw8, n, total_rows):
    d = values.shape[1]
    base, extra = _worker_split(total_rows)
    stride = (n // NS) // 8 * 8
    span = n - (NS - 1) * stride
    assert span % 8 == 0 and span >= stride
    mesh = plsc.VectorSubcoreMesh(
        core_axis_name="c", subcore_axis_name="s", num_cores=NC, num_subcores=NS
    )

    @functools.partial(
        pl.kernel,
        out_type=jax.ShapeDtypeStruct((NC, n, d), jnp.float32),
        mesh=mesh,
        scratch_types=[
            pltpu.VMEM_SHARED((n, d), jnp.float32),
            pltpu.VMEM((rpw8, ROW), jnp.int32),
            [pltpu.VMEM((ROW, d), jnp.float32) for _ in range(2)],
            [pltpu.SemaphoreType.DMA for _ in range(2)],
        ],
    )
    def k(val_hbm, idx_hbm, zero_hbm, out_hbm, acc, idx_v, bufs, sems):
        nbuf = len(bufs)
        c = lax.axis_index("c")
        s = lax.axis_index("s")
        w = s * NC + c
        start = w * base + jnp.minimum(w, extra)
        cnt = base + jnp.where(w < extra, 1, 0)

        pltpu.sync_copy(zero_hbm.at[pl.ds(s * stride, span)], acc.at[pl.ds(s * stride, span)])
        pltpu.sync_copy(idx_hbm.at[w], idx_v)
        plsc.subcore_barrier()

        for p in range(nbuf):
            @pl.when(p < cnt)
            def _():
                pltpu.async_copy(val_hbm.at[pl.ds((start + p) * ROW, ROW)],
                                 bufs[p], sems[p])

        def body(j, _):
            for p in range(nbuf):
                @pl.when(lax.rem(j, nbuf) == p)
                def _():
                    pltpu.make_async_copy(
                        val_hbm.at[pl.ds(0, ROW)], bufs[p], sems[p]).wait()
                    pltpu.sync_copy(bufs[p], acc.at[idx_v.at[j]], add=True)

                    @pl.when(j + nbuf < cnt)
                    def _():
                        pltpu.async_copy(
                            val_hbm.at[pl.ds((start + j + nbuf) * ROW, ROW)],
                            bufs[p], sems[p])
            return 0

        lax.fori_loop(0, cnt, body, 0)
        plsc.subcore_barrier()
        pltpu.sync_copy(acc.at[pl.ds(s * stride, span)], out_hbm.at[c, pl.ds(s * stride, span)])

    zeros = jnp.zeros((n, d), jnp.float32)
    return k(values, idx_slabs, zeros)


def _node_mlp_body(nf_ref, p0_ref, p1_ref, w0_ref, b0_ref, w1_ref, b1_ref,
                   w2_ref, b2_ref, g_ref, bt_ref, out_ref):
    nf = nf_ref[...]
    seg = p0_ref[0] + p1_ref[0]
    w0 = w0_ref[...]
    d = nf.shape[1]
    h = (jnp.dot(nf, w0[0:d], preferred_element_type=jnp.float32)
         + jnp.dot(seg, w0[d:2 * d], preferred_element_type=jnp.float32)
         + b0_ref[...])
    h = jnp.maximum(h, 0.0)
    h = jnp.maximum(jnp.dot(h, w1_ref[...], preferred_element_type=jnp.float32)
                    + b1_ref[...], 0.0)
    h = jnp.dot(h, w2_ref[...], preferred_element_type=jnp.float32) + b2_ref[...]
    h = _ln_affine(h, g_ref[...], bt_ref[...])
    out_ref[...] = h + nf


def _tc_node_mlp(node_features, partials, node_params, block_n):
    n, d = node_features.shape
    nb = n // block_n
    w0, b0, w1, b1, w2, b2, gamma, beta = node_params
    vec = lambda v: v.reshape(1, d)

    return pl.pallas_call(
        _node_mlp_body,
        grid=(nb,),
        in_specs=[
            pl.BlockSpec((block_n, d), lambda i: (i, 0)),
            pl.BlockSpec((1, block_n, d), lambda i: (0, i, 0)),
            pl.BlockSpec((1, block_n, d), lambda i: (1, i, 0)),
            pl.BlockSpec((2 * d, d), lambda i: (0, 0)),
            pl.BlockSpec((1, d), lambda i: (0, 0)),
            pl.BlockSpec((d, d), lambda i: (0, 0)),
            pl.BlockSpec((1, d), lambda i: (0, 0)),
            pl.BlockSpec((d, d), lambda i: (0, 0)),
            pl.BlockSpec((1, d), lambda i: (0, 0)),
            pl.BlockSpec((1, d), lambda i: (0, 0)),
            pl.BlockSpec((1, d), lambda i: (0, 0)),
        ],
        out_specs=pl.BlockSpec((block_n, d), lambda i: (i, 0)),
        out_shape=jax.ShapeDtypeStruct((n, d), jnp.float32),
        compiler_params=pltpu.CompilerParams(
            dimension_semantics=("arbitrary",),
        ),
    )(node_features, partials, partials, w0, vec(b0), w1, vec(b1), w2, vec(b2),
      vec(gamma), vec(beta))


def kernel(node_features, edge_features, senders, receivers, edge_params, node_params):
    n, d = node_features.shape
    e = edge_features.shape[0]
    assert e % ROW == 0 and n % NS == 0

    r_gather = 2 * e // ROW
    idx_all = jnp.concatenate([senders, receivers]).reshape(r_gather, ROW)
    g_slabs, g_rpw8 = _worker_slabs(idx_all, r_gather)
    gathered = _sc_gather(node_features, g_slabs, g_rpw8, r_gather)

    mlp_out, new_edge = _tc_edge_mlp(gathered, edge_features, edge_params,
                                     block_e=2000)

    r_scatter = e // ROW
    s_slabs, s_rpw8 = _worker_slabs(receivers.reshape(r_scatter, ROW), r_scatter)
    partials = _sc_scatter(mlp_out, s_slabs, s_rpw8, n, r_scatter)

    new_node = _tc_node_mlp(node_features, partials, node_params, block_n=1000)
    return (new_node, new_edge)


# speedup vs baseline: 1.0692x; 1.0692x over previous
import functools

import jax
import jax.numpy as jnp
import numpy as np
from jax import lax
from jax.experimental import pallas as pl
from jax.experimental.pallas import tpu as pltpu
from jax.experimental.pallas import tpu_sc as plsc

NC = 2
NS = 16
NW = NC * NS
ROW = 128


def _worker_split(total_rows):
    base = total_rows // NW
    extra = total_rows % NW
    return base, extra


def _worker_slabs(idx_rows, total_rows):
    base, extra = _worker_split(total_rows)
    rpw8 = -(-(base + 1) // 8) * 8
    starts = np.array([w * base + min(w, extra) for w in range(NW)])
    rows = np.minimum(starts[:, None] + np.arange(rpw8)[None, :], total_rows - 1)
    return jnp.take(idx_rows, jnp.asarray(rows), axis=0), rpw8


def _sc_gather(table, idx_slabs, rpw8, total_rows):
    n, d = table.shape
    base, extra = _worker_split(total_rows)
    mesh = plsc.VectorSubcoreMesh(
        core_axis_name="c", subcore_axis_name="s", num_cores=NC, num_subcores=NS
    )

    nbuf = 4

    @functools.partial(
        pl.kernel,
        out_type=jax.ShapeDtypeStruct((total_rows * ROW, d), jnp.float32),
        mesh=mesh,
        scratch_types=[
            pltpu.VMEM((rpw8, ROW), jnp.int32),
            [pltpu.VMEM((ROW, d), jnp.float32) for _ in range(nbuf)],
            [pltpu.SemaphoreType.DMA for _ in range(nbuf)],
        ],
    )
    def k(table_hbm, idx_hbm, out_hbm, idx_v, bufs, gsems):
        c = lax.axis_index("c")
        s = lax.axis_index("s")
        w = s * NC + c
        start = w * base + jnp.minimum(w, extra)
        cnt = base + jnp.where(w < extra, 1, 0)
        pltpu.sync_copy(idx_hbm.at[w], idx_v)

        for p in range(nbuf):
            @pl.when(p < cnt)
            def _():
                pltpu.async_copy(table_hbm.at[idx_v.at[p]], bufs[p], gsems[p])

        def body(j, _):
            for p in range(nbuf):
                @pl.when(lax.rem(j, nbuf) == p)
                def _():
                    pltpu.make_async_copy(
                        table_hbm.at[idx_v.at[0]], bufs[p], gsems[p]).wait()
                    pltpu.sync_copy(bufs[p], out_hbm.at[pl.ds((start + j) * ROW, ROW)])

                    @pl.when(j + nbuf < cnt)
                    def _():
                        pltpu.async_copy(table_hbm.at[idx_v.at[j + nbuf]],
                                         bufs[p], gsems[p])
            return 0

        lax.fori_loop(0, cnt, body, 0)

    return k(table, idx_slabs)


def _ln_affine(h, gamma, beta, eps=1e-5):
    mu = jnp.mean(h, axis=-1, keepdims=True)
    xc = h - mu
    var = jnp.mean(xc * xc, axis=-1, keepdims=True)
    return xc * lax.rsqrt(var + eps) * gamma + beta


def _edge_mlp_body(gs_ref, gr_ref, e_ref, w0_ref, b0_ref, w1_ref, b1_ref,
                   w2_ref, b2_ref, g_ref, bt_ref, prev_ref, mlp_ref, edge_ref):
    s = gs_ref[0]
    r = gr_ref[0]
    e = e_ref[...]
    w0 = w0_ref[...]
    d = e.shape[1]
    h = (jnp.dot(s, w0[0:d], preferred_element_type=jnp.float32)
         + jnp.dot(r, w0[d:2 * d], preferred_element_type=jnp.float32)
         + jnp.dot(e, w0[2 * d:3 * d], preferred_element_type=jnp.float32)
         + b0_ref[...])
    h = jnp.maximum(h, 0.0)
    h = jnp.maximum(jnp.dot(h, w1_ref[...], preferred_element_type=jnp.float32)
                    + b1_ref[...], 0.0)
    h = jnp.dot(h, w2_ref[...], preferred_element_type=jnp.float32) + b2_ref[...]
    h = _ln_affine(h, g_ref[...], bt_ref[...])
    mlp_ref[...] = h
    edge_ref[...] = h + e


def _tc_edge_mlp(gathered, edge_features, edge_params, block_e, chunk, prev_buf):
    ec, d = gathered.shape[0] // 2, gathered.shape[1]
    nb = ec // block_e
    off = chunk * nb
    w0, b0, w1, b1, w2, b2, gamma, beta = edge_params
    vec = lambda v: v.reshape(1, d)
    g3 = gathered.reshape(2, ec, d)
    e_total = edge_features.shape[0]
    if prev_buf is None:
        prev_buf = edge_features
        aliases = {}
    else:
        aliases = {11: 1}

    out = pl.pallas_call(
        _edge_mlp_body,
        grid=(nb,),
        in_specs=[
            pl.BlockSpec((1, block_e, d), lambda i: (0, i, 0)),
            pl.BlockSpec((1, block_e, d), lambda i: (1, i, 0)),
            pl.BlockSpec((block_e, d), lambda i: (off + i, 0)),
            pl.BlockSpec((3 * d, d), lambda i: (0, 0)),
            pl.BlockSpec((1, d), lambda i: (0, 0)),
            pl.BlockSpec((d, d), lambda i: (0, 0)),
            pl.BlockSpec((1, d), lambda i: (0, 0)),
            pl.BlockSpec((d, d), lambda i: (0, 0)),
            pl.BlockSpec((1, d), lambda i: (0, 0)),
            pl.BlockSpec((1, d), lambda i: (0, 0)),
            pl.BlockSpec((1, d), lambda i: (0, 0)),
            pl.BlockSpec((8, d), lambda i: (0, 0)),
        ],
        out_specs=[
            pl.BlockSpec((block_e, d), lambda i: (i, 0)),
            pl.BlockSpec((block_e, d), lambda i: (off + i, 0)),
        ],
        out_shape=[
            jax.ShapeDtypeStruct((ec, d), jnp.float32),
            jax.ShapeDtypeStruct((e_total, d), jnp.float32),
        ],
        input_output_aliases=aliases,
        compiler_params=pltpu.CompilerParams(
            dimension_semantics=("arbitrary",),
        ),
    )(g3, g3, edge_features, w0, vec(b0), w1, vec(b1), w2, vec(b2),
      vec(gamma), vec(beta), prev_buf)
    return out


def _sc_scatter(values, idx_slabs, rpw8, n, total_rows):
    d = values.shape[1]
    base, extra = _worker_split(total_rows)
    stride = (n // NS) // 8 * 8
    span = n - (NS - 1) * stride
    assert span % 8 == 0 and span >= stride
    mesh = plsc.VectorSubcoreMesh(
        core_axis_name="c", subcore_axis_name="s", num_cores=NC, num_subcores=NS
    )

    @functools.partial(
        pl.kernel,
        out_type=jax.ShapeDtypeStruct((NC, n, d), jnp.float32),
        mesh=mesh,
        scratch_types=[
            pltpu.VMEM_SHARED((n, d), jnp.float32),
            pltpu.VMEM((rpw8, ROW), jnp.int32),
            [pltpu.VMEM((ROW, d), jnp.float32) for _ in range(2)],
            [pltpu.SemaphoreType.DMA for _ in range(2)],
        ],
    )
    def k(val_hbm, idx_hbm, zero_hbm, out_hbm, acc, idx_v, bufs, sems):
        nbuf = len(bufs)
        c = lax.axis_index("c")
        s = lax.axis_index("s")
        w = s * NC + c
        start = w * base + jnp.minimum(w, extra)
        cnt = base + jnp.where(w < extra, 1, 0)

        pltpu.sync_copy(zero_hbm.at[pl.ds(s * stride, span)], acc.at[pl.ds(s * stride, span)])
        pltpu.sync_copy(idx_hbm.at[w], idx_v)
        plsc.subcore_barrier()

        for p in range(nbuf):
            @pl.when(p < cnt)
            def _():
                pltpu.async_copy(val_hbm.at[pl.ds((start + p) * ROW, ROW)],
                                 bufs[p], sems[p])

        def body(j, _):
            for p in range(nbuf):
                @pl.when(lax.rem(j, nbuf) == p)
                def _():
                    pltpu.make_async_copy(
                        val_hbm.at[pl.ds(0, ROW)], bufs[p], sems[p]).wait()
                    pltpu.sync_copy(bufs[p], acc.at[idx_v.at[j]], add=True)

                    @pl.when(j + nbuf < cnt)
                    def _():
                        pltpu.async_copy(
                            val_hbm.at[pl.ds((start + j + nbuf) * ROW, ROW)],
                            bufs[p], sems[p])
            return 0

        lax.fori_loop(0, cnt, body, 0)
        plsc.subcore_barrier()
        pltpu.sync_copy(acc.at[pl.ds(s * stride, span)], out_hbm.at[c, pl.ds(s * stride, span)])

    zeros = jnp.zeros((n, d), jnp.float32)
    return k(values, idx_slabs, zeros)


def _node_mlp_body(nf_ref, *refs):
    *p_refs, w0_ref, b0_ref, w1_ref, b1_ref, w2_ref, b2_ref, g_ref, bt_ref, out_ref = refs
    nf = nf_ref[...]
    seg = p_refs[0][0]
    for p in p_refs[1:]:
        seg = seg + p[0]
    w0 = w0_ref[...]
    d = nf.shape[1]
    h = (jnp.dot(nf, w0[0:d], preferred_element_type=jnp.float32)
         + jnp.dot(seg, w0[d:2 * d], preferred_element_type=jnp.float32)
         + b0_ref[...])
    h = jnp.maximum(h, 0.0)
    h = jnp.maximum(jnp.dot(h, w1_ref[...], preferred_element_type=jnp.float32)
                    + b1_ref[...], 0.0)
    h = jnp.dot(h, w2_ref[...], preferred_element_type=jnp.float32) + b2_ref[...]
    h = _ln_affine(h, g_ref[...], bt_ref[...])
    out_ref[...] = h + nf


def _tc_node_mlp(node_features, partials, node_params, block_n):
    n, d = node_features.shape
    nb = n // block_n
    w0, b0, w1, b1, w2, b2, gamma, beta = node_params
    vec = lambda v: v.reshape(1, d)

    p_inputs, p_specs = [], []
    for p in partials:
        for core in range(NC):
            p_inputs.append(p)
            p_specs.append(
                pl.BlockSpec((1, block_n, d),
                             functools.partial(lambda c, i: (c, i, 0), core)))

    return pl.pallas_call(
        _node_mlp_body,
        grid=(nb,),
        in_specs=[pl.BlockSpec((block_n, d), lambda i: (i, 0))] + p_specs + [
            pl.BlockSpec((2 * d, d), lambda i: (0, 0)),
            pl.BlockSpec((1, d), lambda i: (0, 0)),
            pl.BlockSpec((d, d), lambda i: (0, 0)),
            pl.BlockSpec((1, d), lambda i: (0, 0)),
            pl.BlockSpec((d, d), lambda i: (0, 0)),
            pl.BlockSpec((1, d), lambda i: (0, 0)),
            pl.BlockSpec((1, d), lambda i: (0, 0)),
            pl.BlockSpec((1, d), lambda i: (0, 0)),
        ],
        out_specs=pl.BlockSpec((block_n, d), lambda i: (i, 0)),
        out_shape=jax.ShapeDtypeStruct((n, d), jnp.float32),
        compiler_params=pltpu.CompilerParams(
            dimension_semantics=("arbitrary",),
        ),
    )(node_features, *p_inputs, w0, vec(b0), w1, vec(b1), w2, vec(b2),
      vec(gamma), vec(beta))


def kernel(node_features, edge_features, senders, receivers, edge_params, node_params):
    n, d = node_features.shape
    e = edge_features.shape[0]
    nch = 4
    block_e = 2000
    ec = e // nch
    assert e % (nch * ROW) == 0 and ec % block_e == 0 and n % NS == 0

    s2 = senders.reshape(nch, ec)
    r2 = receivers.reshape(nch, ec)
    gidx = jnp.concatenate([s2, r2], axis=1)

    rpc_g = 2 * ec // ROW
    rpc_s = ec // ROW
    partials = []
    new_edge = None
    for c in range(nch):
        g_slabs, g_rpw8 = _worker_slabs(gidx[c].reshape(rpc_g, ROW), rpc_g)
        gathered = _sc_gather(node_features, g_slabs, g_rpw8, rpc_g)
        mlp_c, new_edge = _tc_edge_mlp(gathered, edge_features, edge_params,
                                       block_e, c, new_edge)
        s_slabs, s_rpw8 = _worker_slabs(r2[c].reshape(rpc_s, ROW), rpc_s)
        partials.append(_sc_scatter(mlp_c, s_slabs, s_rpw8, n, rpc_s))

    new_node = _tc_node_mlp(node_features, partials, node_params, block_n=1000)
    return (new_node, new_edge)


# speedup vs baseline: 1.3080x; 1.2234x over previous
import functools

import jax
import jax.numpy as jnp
import numpy as np
from jax import lax
from jax.experimental import pallas as pl
from jax.experimental.pallas import tpu as pltpu
from jax.experimental.pallas import tpu_sc as plsc

NC = 2
NS = 16
NW = NC * NS
ROW = 128


def _worker_split(total_rows):
    base = total_rows // NW
    extra = total_rows % NW
    return base, extra


def _worker_slabs(idx_rows, total_rows):
    base, extra = _worker_split(total_rows)
    rpw8 = -(-(base + 1) // 8) * 8
    starts = np.array([w * base + min(w, extra) for w in range(NW)])
    rows = np.minimum(starts[:, None] + np.arange(rpw8)[None, :], total_rows - 1)
    return jnp.take(idx_rows, jnp.asarray(rows), axis=0), rpw8


def _project_body(nf_ref, w_ref, out_ref):
    out_ref[...] = jnp.dot(nf_ref[...], w_ref[0],
                           preferred_element_type=jnp.float32)


def _tc_project(node_features, w0sr, block_n):
    n, d = node_features.shape
    nb = n // block_n
    return pl.pallas_call(
        _project_body,
        grid=(2, nb),
        in_specs=[
            pl.BlockSpec((block_n, d), lambda p, i: (i, 0)),
            pl.BlockSpec((1, d, d), lambda p, i: (p, 0, 0)),
        ],
        out_specs=pl.BlockSpec((block_n, d), lambda p, i: (p * nb + i, 0)),
        out_shape=jax.ShapeDtypeStruct((2 * n, d), jnp.float32),
        compiler_params=pltpu.CompilerParams(
            dimension_semantics=("arbitrary", "arbitrary"),
        ),
    )(node_features, w0sr)


def _sc_gather_add(table, sidx_slabs, ridx_slabs, rpw8, total_rows):
    d = table.shape[1]
    base, extra = _worker_split(total_rows)
    mesh = plsc.VectorSubcoreMesh(
        core_axis_name="c", subcore_axis_name="s", num_cores=NC, num_subcores=NS
    )

    @functools.partial(
        pl.kernel,
        out_type=jax.ShapeDtypeStruct((total_rows * ROW, d), jnp.float32),
        mesh=mesh,
        scratch_types=[
            pltpu.VMEM((rpw8, ROW), jnp.int32),
            pltpu.VMEM((rpw8, ROW), jnp.int32),
            [pltpu.VMEM((ROW, d), jnp.float32) for _ in range(4)],
            [pltpu.SemaphoreType.DMA for _ in range(4)],
        ],
    )
    def k(table_hbm, sidx_hbm, ridx_hbm, out_hbm, sidx_v, ridx_v, bufs, sems):
        c = lax.axis_index("c")
        s = lax.axis_index("s")
        w = s * NC + c
        start = w * base + jnp.minimum(w, extra)
        cnt = base + jnp.where(w < extra, 1, 0)
        pltpu.sync_copy(sidx_hbm.at[w], sidx_v)
        pltpu.sync_copy(ridx_hbm.at[w], ridx_v)

        def issue(j, p):
            pltpu.async_copy(table_hbm.at[sidx_v.at[j]], bufs[2 * p], sems[2 * p])
            pltpu.async_copy(table_hbm.at[ridx_v.at[j]], bufs[2 * p + 1],
                             sems[2 * p + 1])

        @pl.when(cnt > 0)
        def _():
            issue(0, 0)

        def body(j, _):
            for p in range(2):
                @pl.when(lax.rem(j, 2) == p)
                def _():
                    a, b = bufs[2 * p], bufs[2 * p + 1]
                    pltpu.make_async_copy(
                        table_hbm.at[sidx_v.at[0]], a, sems[2 * p]).wait()
                    pltpu.make_async_copy(
                        table_hbm.at[ridx_v.at[0]], b, sems[2 * p + 1]).wait()

                    @pl.when(j + 1 < cnt)
                    def _():
                        issue(j + 1, 1 - p)

                    def add_row(r, _):
                        for cb in range(d // 16):
                            a[r, pl.ds(cb * 16, 16)] = (
                                a[r, pl.ds(cb * 16, 16)] + b[r, pl.ds(cb * 16, 16)])
                        return 0

                    lax.fori_loop(0, ROW, add_row, 0)
                    pltpu.sync_copy(a, out_hbm.at[pl.ds((start + j) * ROW, ROW)])
            return 0

        lax.fori_loop(0, cnt, body, 0)

    return k(table, sidx_slabs, ridx_slabs)


def _sc_gather(table, idx_slabs, rpw8, total_rows):
    n, d = table.shape
    base, extra = _worker_split(total_rows)
    mesh = plsc.VectorSubcoreMesh(
        core_axis_name="c", subcore_axis_name="s", num_cores=NC, num_subcores=NS
    )

    nbuf = 4

    @functools.partial(
        pl.kernel,
        out_type=jax.ShapeDtypeStruct((total_rows * ROW, d), jnp.float32),
        mesh=mesh,
        scratch_types=[
            pltpu.VMEM((rpw8, ROW), jnp.int32),
            [pltpu.VMEM((ROW, d), jnp.float32) for _ in range(nbuf)],
            [pltpu.SemaphoreType.DMA for _ in range(nbuf)],
        ],
    )
    def k(table_hbm, idx_hbm, out_hbm, idx_v, bufs, gsems):
        c = lax.axis_index("c")
        s = lax.axis_index("s")
        w = s * NC + c
        start = w * base + jnp.minimum(w, extra)
        cnt = base + jnp.where(w < extra, 1, 0)
        pltpu.sync_copy(idx_hbm.at[w], idx_v)

        for p in range(nbuf):
            @pl.when(p < cnt)
            def _():
                pltpu.async_copy(table_hbm.at[idx_v.at[p]], bufs[p], gsems[p])

        def body(j, _):
            for p in range(nbuf):
                @pl.when(lax.rem(j, nbuf) == p)
                def _():
                    pltpu.make_async_copy(
                        table_hbm.at[idx_v.at[0]], bufs[p], gsems[p]).wait()
                    pltpu.sync_copy(bufs[p], out_hbm.at[pl.ds((start + j) * ROW, ROW)])

                    @pl.when(j + nbuf < cnt)
                    def _():
                        pltpu.async_copy(table_hbm.at[idx_v.at[j + nbuf]],
                                         bufs[p], gsems[p])
            return 0

        lax.fori_loop(0, cnt, body, 0)

    return k(table, idx_slabs)


def _ln_affine(h, gamma, beta, eps=1e-5):
    mu = jnp.mean(h, axis=-1, keepdims=True)
    xc = h - mu
    var = jnp.mean(xc * xc, axis=-1, keepdims=True)
    return xc * lax.rsqrt(var + eps) * gamma + beta


def _edge_mlp_body(gsum_ref, e_ref, w0e_ref, b0_ref, w1_ref, b1_ref,
                   w2_ref, b2_ref, g_ref, bt_ref, prev_ref, mlp_ref, edge_ref):
    e = e_ref[...]
    h = (gsum_ref[...]
         + jnp.dot(e, w0e_ref[...], preferred_element_type=jnp.float32)
         + b0_ref[...])
    h = jnp.maximum(h, 0.0)
    h = jnp.maximum(jnp.dot(h, w1_ref[...], preferred_element_type=jnp.float32)
                    + b1_ref[...], 0.0)
    h = jnp.dot(h, w2_ref[...], preferred_element_type=jnp.float32) + b2_ref[...]
    h = _ln_affine(h, g_ref[...], bt_ref[...])
    mlp_ref[...] = h
    edge_ref[...] = h + e


def _tc_edge_mlp(gsum, edge_features, w0e, edge_params, block_e, chunk, prev_buf):
    ec, d = gsum.shape
    nb = ec // block_e
    off = chunk * nb
    _, b0, w1, b1, w2, b2, gamma, beta = edge_params
    vec = lambda v: v.reshape(1, d)
    e_total = edge_features.shape[0]
    if prev_buf is None:
        prev_buf = edge_features
        aliases = {}
    else:
        aliases = {10: 1}

    out = pl.pallas_call(
        _edge_mlp_body,
        grid=(nb,),
        in_specs=[
            pl.BlockSpec((block_e, d), lambda i: (i, 0)),
            pl.BlockSpec((block_e, d), lambda i: (off + i, 0)),
            pl.BlockSpec((d, d), lambda i: (0, 0)),
            pl.BlockSpec((1, d), lambda i: (0, 0)),
            pl.BlockSpec((d, d), lambda i: (0, 0)),
            pl.BlockSpec((1, d), lambda i: (0, 0)),
            pl.BlockSpec((d, d), lambda i: (0, 0)),
            pl.BlockSpec((1, d), lambda i: (0, 0)),
            pl.BlockSpec((1, d), lambda i: (0, 0)),
            pl.BlockSpec((1, d), lambda i: (0, 0)),
            pl.BlockSpec((8, d), lambda i: (0, 0)),
        ],
        out_specs=[
            pl.BlockSpec((block_e, d), lambda i: (i, 0)),
            pl.BlockSpec((block_e, d), lambda i: (off + i, 0)),
        ],
        out_shape=[
            jax.ShapeDtypeStruct((ec, d), jnp.float32),
            jax.ShapeDtypeStruct((e_total, d), jnp.float32),
        ],
        input_output_aliases=aliases,
        compiler_params=pltpu.CompilerParams(
            dimension_semantics=("arbitrary",),
        ),
    )(gsum, edge_features, w0e, vec(b0), w1, vec(b1), w2, vec(b2),
      vec(gamma), vec(beta), prev_buf)
    return out


def _sc_scatter(values_list, slabs_list, rpw8, n, total_rows):
    kk = len(values_list)
    d = values_list[0].shape[1]
    base, extra = _worker_split(total_rows)
    stride = (n // NS) // 8 * 8
    span = n - (NS - 1) * stride
    assert span % 8 == 0 and span >= stride
    mesh = plsc.VectorSubcoreMesh(
        core_axis_name="c", subcore_axis_name="s", num_cores=NC, num_subcores=NS
    )

    @functools.partial(
        pl.kernel,
        out_type=jax.ShapeDtypeStruct((NC, n, d), jnp.float32),
        mesh=mesh,
        scratch_types=[
            pltpu.VMEM_SHARED((n, d), jnp.float32),
            pltpu.VMEM((rpw8, ROW), jnp.int32),
            [pltpu.VMEM((ROW, d), jnp.float32) for _ in range(2)],
            [pltpu.SemaphoreType.DMA for _ in range(2)],
        ],
    )
    def k(*refs):
        v_hbms = refs[:kk]
        s_hbms = refs[kk:2 * kk]
        zero_hbm = refs[2 * kk]
        acc, idx_v, bufs, sems = refs[2 * kk + 2:]
        out_hbm = refs[2 * kk + 1]
        nbuf = len(bufs)
        c = lax.axis_index("c")
        s = lax.axis_index("s")
        w = s * NC + c
        start = w * base + jnp.minimum(w, extra)
        cnt = base + jnp.where(w < extra, 1, 0)

        pltpu.sync_copy(zero_hbm.at[pl.ds(s * stride, span)], acc.at[pl.ds(s * stride, span)])
        plsc.subcore_barrier()

        for val_hbm, slab_hbm in zip(v_hbms, s_hbms):
            pltpu.sync_copy(slab_hbm.at[w], idx_v)

            for p in range(2):
                @pl.when(p < cnt)
                def _():
                    pltpu.async_copy(val_hbm.at[pl.ds((start + p) * ROW, ROW)],
                                     bufs[p], sems[p])

            def body(j, _):
                for p in range(nbuf):
                    @pl.when(lax.rem(j, nbuf) == p)
                    def _():
                        pltpu.make_async_copy(
                            val_hbm.at[pl.ds(0, ROW)], bufs[p], sems[p]).wait()
                        pltpu.sync_copy(bufs[p], acc.at[idx_v.at[j]], add=True)

                        @pl.when(j + nbuf < cnt)
                        def _():
                            pltpu.async_copy(
                                val_hbm.at[pl.ds((start + j + nbuf) * ROW, ROW)],
                                bufs[p], sems[p])
                return 0

            lax.fori_loop(0, cnt, body, 0)

        plsc.subcore_barrier()
        pltpu.sync_copy(acc.at[pl.ds(s * stride, span)], out_hbm.at[c, pl.ds(s * stride, span)])

    zeros = jnp.zeros((n, d), jnp.float32)
    return k(*values_list, *slabs_list, zeros)


def _node_mlp_body(nf_ref, *refs):
    *p_refs, w0_ref, b0_ref, w1_ref, b1_ref, w2_ref, b2_ref, g_ref, bt_ref, out_ref = refs
    nf = nf_ref[...]
    seg = p_refs[0][0]
    for p in p_refs[1:]:
        seg = seg + p[0]
    w0 = w0_ref[...]
    d = nf.shape[1]
    h = (jnp.dot(nf, w0[0:d], preferred_element_type=jnp.float32)
         + jnp.dot(seg, w0[d:2 * d], preferred_element_type=jnp.float32)
         + b0_ref[...])
    h = jnp.maximum(h, 0.0)
    h = jnp.maximum(jnp.dot(h, w1_ref[...], preferred_element_type=jnp.float32)
                    + b1_ref[...], 0.0)
    h = jnp.dot(h, w2_ref[...], preferred_element_type=jnp.float32) + b2_ref[...]
    h = _ln_affine(h, g_ref[...], bt_ref[...])
    out_ref[...] = h + nf


def _tc_node_mlp(node_features, partials, node_params, block_n):
    n, d = node_features.shape
    nb = n // block_n
    w0, b0, w1, b1, w2, b2, gamma, beta = node_params
    vec = lambda v: v.reshape(1, d)

    p_inputs, p_specs = [], []
    for p in partials:
        for core in range(NC):
            p_inputs.append(p)
            p_specs.append(
                pl.BlockSpec((1, block_n, d),
                             functools.partial(lambda c, i: (c, i, 0), core)))

    return pl.pallas_call(
        _node_mlp_body,
        grid=(nb,),
        in_specs=[pl.BlockSpec((block_n, d), lambda i: (i, 0))] + p_specs + [
            pl.BlockSpec((2 * d, d), lambda i: (0, 0)),
            pl.BlockSpec((1, d), lambda i: (0, 0)),
            pl.BlockSpec((d, d), lambda i: (0, 0)),
            pl.BlockSpec((1, d), lambda i: (0, 0)),
            pl.BlockSpec((d, d), lambda i: (0, 0)),
            pl.BlockSpec((1, d), lambda i: (0, 0)),
            pl.BlockSpec((1, d), lambda i: (0, 0)),
            pl.BlockSpec((1, d), lambda i: (0, 0)),
        ],
        out_specs=pl.BlockSpec((block_n, d), lambda i: (i, 0)),
        out_shape=jax.ShapeDtypeStruct((n, d), jnp.float32),
        compiler_params=pltpu.CompilerParams(
            dimension_semantics=("arbitrary",),
        ),
    )(node_features, *p_inputs, w0, vec(b0), w1, vec(b1), w2, vec(b2),
      vec(gamma), vec(beta))


def kernel(node_features, edge_features, senders, receivers, edge_params, node_params):
    n, d = node_features.shape
    e = edge_features.shape[0]
    nch = 4
    block_e = 2000
    ec = e // nch
    assert e % (nch * ROW) == 0 and ec % block_e == 0 and n % NS == 0

    s2 = senders.reshape(nch, ec)
    r2 = receivers.reshape(nch, ec)
    ro2 = r2 + n

    w0 = edge_params[0]
    proj_table = _tc_project(node_features, w0[: 2 * d].reshape(2, d, d), 1000)
    w0e = w0[2 * d:]

    rpc = ec // ROW
    mlps, sc_slabs = [], []
    new_edge = None
    for c in range(nch):
        s_slabs, rpw8 = _worker_slabs(s2[c].reshape(rpc, ROW), rpc)
        r_slabs, _ = _worker_slabs(ro2[c].reshape(rpc, ROW), rpc)
        gsum = _sc_gather_add(proj_table, s_slabs, r_slabs, rpw8, rpc)
        mlp_c, new_edge = _tc_edge_mlp(gsum, edge_features, w0e, edge_params,
                                       block_e, c, new_edge)
        mlps.append(mlp_c)
        slabs, s_rpw8 = _worker_slabs(r2[c].reshape(rpc, ROW), rpc)
        sc_slabs.append(slabs)

    partials = []
    for h in range(0, nch, 2):
        partials.append(_sc_scatter(mlps[h:h + 2], sc_slabs[h:h + 2],
                                    s_rpw8, n, rpc))

    new_node = _tc_node_mlp(node_features, partials, node_params, block_n=1000)
    return (new_node, new_edge)
